# baseline (speedup 1.0000x reference)
import functools

import jax
import jax.numpy as jnp
from jax import lax
from jax.experimental import pallas as pl
from jax.experimental.pallas import tpu as pltpu
from jax.experimental.pallas import tpu_sc as plsc

_NC = 2
_NS = 16
_NW = _NC * _NS


def _make_coords_gather(b, c_dim):
    b_per_w = b // _NW
    mesh = plsc.VectorSubcoreMesh(
        core_axis_name="c", subcore_axis_name="s", num_cores=_NC,
        num_subcores=_NS)

    @functools.partial(
        pl.kernel,
        mesh=mesh,
        out_type=jax.ShapeDtypeStruct((b, c_dim), jnp.float32),
        scratch_types=[
            pltpu.VMEM((b_per_w,), jnp.int32),
            pltpu.VMEM((b_per_w, c_dim), jnp.float32),
            pltpu.SemaphoreType.DMA,
        ],
    )
    def coords_kernel(coords_hbm, idx_hbm, lab_hbm, idx_v, crows_v, csem):
        wid = lax.axis_index("s") * _NC + lax.axis_index("c")
        base = wid * b_per_w
        pltpu.sync_copy(idx_hbm.at[pl.ds(base, b_per_w)], idx_v)
        pltpu.async_copy(coords_hbm.at[idx_v], crows_v, csem).wait()
        pltpu.sync_copy(crows_v, lab_hbm.at[pl.ds(base, b_per_w)])

    return coords_kernel


def _make_row_gather(n, h, w, b):
    def row_kernel(idx_smem, spec_any, out_any, sem):
        def body(i, _):
            row = idx_smem[i]
            pltpu.make_async_copy(
                spec_any.at[pl.ds(row, 1)],
                out_any.at[pl.ds(i, 1)], sem).start()
            return ()

        lax.fori_loop(0, b, body, ())
        pltpu.make_async_copy(
            spec_any.at[pl.ds(0, b)], out_any, sem).wait()

    return pl.pallas_call(
        row_kernel,
        out_shape=jax.ShapeDtypeStruct((b, h, w), jnp.float32),
        in_specs=[
            pl.BlockSpec(memory_space=pltpu.SMEM),
            pl.BlockSpec(memory_space=pl.ANY),
        ],
        out_specs=pl.BlockSpec(memory_space=pl.ANY),
        scratch_shapes=[pltpu.SemaphoreType.DMA],
    )


def kernel(spectrograms, coords, indices):
    n, h, w = spectrograms.shape
    b = indices.shape[0]
    c_dim = coords.shape[1]
    c_pad = 128
    coords_p = jnp.pad(coords, ((0, 0), (0, c_pad - c_dim)))
    labels = _make_coords_gather(b, c_pad)(coords_p, indices)
    samples = _make_row_gather(n, h, w, b)(indices, spectrograms)
    return samples[:, None, :, :], labels[:, :c_dim]


# speedup vs baseline: 13.3848x; 13.3848x over previous
import functools

import jax
import jax.numpy as jnp
from jax import lax
from jax.experimental import pallas as pl
from jax.experimental.pallas import tpu as pltpu
from jax.experimental.pallas import tpu_sc as plsc

_NC = 2
_NS = 16
_NW = _NC * _NS


def _make_gather(nh, h, w, b, c_dim, grp):
    b_per_w = b // _NW
    ngrp = b_per_w // grp
    mesh = plsc.VectorSubcoreMesh(
        core_axis_name="c", subcore_axis_name="s", num_cores=_NC,
        num_subcores=_NS)

    @functools.partial(
        pl.kernel,
        mesh=mesh,
        out_type=[
            jax.ShapeDtypeStruct((b * h, w), jnp.float32),
            jax.ShapeDtypeStruct((b, c_dim), jnp.float32),
        ],
        scratch_types=[
            pltpu.VMEM((b_per_w,), jnp.int32),
            pltpu.VMEM_SHARED((_NS, b_per_w), jnp.int32),
            pltpu.SMEM((b_per_w,), jnp.int32),
            pltpu.VMEM((2, grp * h, w), jnp.float32),
            pltpu.VMEM((b_per_w, c_dim), jnp.float32),
            pltpu.SemaphoreType.DMA,
            pltpu.SemaphoreType.DMA,
            pltpu.SemaphoreType.DMA,
            pltpu.SemaphoreType.DMA,
            pltpu.SemaphoreType.DMA,
        ],
    )
    def gather_kernel(spec_hbm, coords_hbm, idx_hbm, out_hbm, lab_hbm,
                      idx_v, idx_sh, idx_s, rows_v, crows_v, gsem0, gsem1,
                      ssem0, ssem1, csem):
        wid = lax.axis_index("s") * _NC + lax.axis_index("c")
        sid = lax.axis_index("s")
        base = wid * b_per_w

        pltpu.sync_copy(idx_hbm.at[pl.ds(base, b_per_w)], idx_v)
        pltpu.sync_copy(idx_v, idx_sh.at[sid])
        pltpu.sync_copy(idx_sh.at[sid], idx_s)

        ccopy = pltpu.async_copy(coords_hbm.at[idx_v], crows_v, csem)

        gsems = (gsem0, gsem1)
        ssems = (ssem0, ssem1)

        def start_group(g):
            buf = g % 2
            copies = []
            for k in range(grp):
                row = idx_s[g * grp + k]
                copies.append(pltpu.async_copy(
                    spec_hbm.at[pl.ds(row * h, h)],
                    rows_v.at[buf, pl.ds(k * h, h)], gsems[buf]))
            return copies

        gathers = [start_group(0), start_group(1)]
        scatters = [None, None]
        for g in range(ngrp):
            buf = g % 2
            for cp in gathers[buf]:
                cp.wait()
            scatters[buf] = pltpu.async_copy(
                rows_v.at[buf],
                out_hbm.at[pl.ds((base + g * grp) * h, grp * h)],
                ssems[buf])
            if g + 2 < ngrp:
                scatters[buf].wait()
                gathers[buf] = start_group(g + 2)
        for g in (ngrp - 2, ngrp - 1):
            if g >= 0 and scatters[g % 2] is not None:
                scatters[g % 2].wait()

        ccopy.wait()
        pltpu.sync_copy(crows_v, lab_hbm.at[pl.ds(base, b_per_w)])

    return gather_kernel


def kernel(spectrograms, coords, indices):
    n, h, w = spectrograms.shape
    b = indices.shape[0]
    c_dim = coords.shape[1]
    spec2d = spectrograms.reshape(n * h, w)
    c_pad = 128
    coords_p = jnp.pad(coords, ((0, 0), (0, c_pad - c_dim)))
    samples, labels = _make_gather(n * h, h, w, b, c_pad, 4)(
        spec2d, coords_p, indices)
    return samples.reshape(b, 1, h, w), labels[:, :c_dim]


# speedup vs baseline: 13.8520x; 1.0349x over previous
import functools

import jax
import jax.numpy as jnp
from jax import lax
from jax.experimental import pallas as pl
from jax.experimental.pallas import tpu as pltpu
from jax.experimental.pallas import tpu_sc as plsc

_NC = 2
_NS = 16
_NW = _NC * _NS


def _mesh():
    return plsc.VectorSubcoreMesh(
        core_axis_name="c", subcore_axis_name="s", num_cores=_NC,
        num_subcores=_NS)


def _make_coords_gather(b, c_dim):
    b_per_w = b // _NW

    @functools.partial(
        pl.kernel,
        mesh=_mesh(),
        out_type=jax.ShapeDtypeStruct((b, c_dim), jnp.float32),
        scratch_types=[
            pltpu.VMEM((b_per_w,), jnp.int32),
            pltpu.VMEM((b_per_w, c_dim), jnp.float32),
            pltpu.SemaphoreType.DMA,
        ],
    )
    def coords_kernel(coords_hbm, idx_hbm, lab_hbm, idx_v, crows_v, csem):
        wid = lax.axis_index("s") * _NC + lax.axis_index("c")
        base = wid * b_per_w
        pltpu.sync_copy(idx_hbm.at[pl.ds(base, b_per_w)], idx_v)
        pltpu.async_copy(coords_hbm.at[idx_v], crows_v, csem).wait()
        pltpu.sync_copy(crows_v, lab_hbm.at[pl.ds(base, b_per_w)])

    return coords_kernel


def _make_spec_gather(n_rows, d, b, chunk, nbuf):
    b_per_w = b // _NW
    nch = b_per_w // chunk

    @functools.partial(
        pl.kernel,
        mesh=_mesh(),
        out_type=jax.ShapeDtypeStruct((b, d), jnp.float32),
        scratch_types=[
            pltpu.VMEM((b_per_w,), jnp.int32),
            pltpu.VMEM((nbuf, chunk, d), jnp.float32),
        ]
        + [pltpu.SemaphoreType.DMA] * (2 * nbuf),
    )
    def gather_kernel(spec_hbm, idx_hbm, out_hbm, idx_v, rows_v, *sems):
        gsems = sems[:nbuf]
        ssems = sems[nbuf:]
        wid = lax.axis_index("s") * _NC + lax.axis_index("c")
        base = wid * b_per_w

        pltpu.sync_copy(idx_hbm.at[pl.ds(base, b_per_w)], idx_v)

        def start_gather(c):
            buf = c % nbuf
            return pltpu.async_copy(
                spec_hbm.at[idx_v.at[pl.ds(c * chunk, chunk)]],
                rows_v.at[buf], gsems[buf])

        gathers = [None] * nbuf
        for c in range(min(nbuf, nch)):
            gathers[c] = start_gather(c)
        scatters = [None] * nbuf
        for c in range(nch):
            buf = c % nbuf
            gathers[buf].wait()
            scatters[buf] = pltpu.async_copy(
                rows_v.at[buf],
                out_hbm.at[pl.ds(base + c * chunk, chunk)], ssems[buf])
            if c + nbuf < nch:
                scatters[buf].wait()
                gathers[buf] = start_gather(c + nbuf)
        for c in range(max(0, nch - nbuf), nch):
            if scatters[c % nbuf] is not None:
                scatters[c % nbuf].wait()

    return gather_kernel


def kernel(spectrograms, coords, indices):
    n, h, w = spectrograms.shape
    d = h * w
    b = indices.shape[0]
    c_dim = coords.shape[1]
    c_pad = 128
    coords_p = jnp.pad(coords, ((0, 0), (0, c_pad - c_dim)))
    labels = _make_coords_gather(b, c_pad)(coords_p, indices)
    spec2d = spectrograms.reshape(n, d)
    samples = _make_spec_gather(n, d, b, 8, 3)(spec2d, indices)
    return samples.reshape(b, 1, h, w), labels[:, :c_dim]
